# baseline (speedup 1.0000x reference)
import functools

import jax
import jax.numpy as jnp
from jax import lax
from jax.experimental import pallas as pl
from jax.experimental.pallas import tpu as pltpu
from jax.experimental.pallas import tpu_sc as plsc

N = 10000
E = 320000
D_IN = 128
D1 = 256
D2 = 128

NC = 2
NS = 16
NW = NC * NS

N_PAD = 10240
ROWS_PER_TILE = N_PAD // NS
CHUNK = 128
E_PAD = 327680
CHUNKS_PER_TILE = E_PAD // (NW * CHUNK)
DEG_W = 128

_mesh = plsc.VectorSubcoreMesh(
    core_axis_name="c", subcore_axis_name="s", num_cores=NC, num_subcores=NS)


@functools.partial(
    pl.kernel,
    out_type=jax.ShapeDtypeStruct((NW, ROWS_PER_TILE, DEG_W), jnp.float32),
    mesh=_mesh,
    scratch_types=[
        pltpu.VMEM((CHUNKS_PER_TILE, CHUNK), jnp.int32),
        pltpu.VMEM((CHUNK, DEG_W), jnp.float32),
        pltpu.VMEM_SHARED((N_PAD, DEG_W), jnp.float32),
    ],
)
def _sc_degree(dst_hbm, zeros_hbm, ones_hbm, out_hbm, dst_v, ones_v, deg_sh):
    cid = lax.axis_index("c")
    sid = lax.axis_index("s")
    wid = sid * NC + cid
    nbase = sid * ROWS_PER_TILE
    pltpu.sync_copy(dst_hbm.at[pl.ds(wid * CHUNKS_PER_TILE, CHUNKS_PER_TILE)],
                    dst_v)
    pltpu.sync_copy(ones_hbm, ones_v)
    pltpu.sync_copy(zeros_hbm.at[pl.ds(nbase, ROWS_PER_TILE)],
                    deg_sh.at[pl.ds(nbase, ROWS_PER_TILE)])
    plsc.subcore_barrier()

    @pl.loop(0, CHUNKS_PER_TILE)
    def _(j):
        pltpu.sync_copy(ones_v, deg_sh.at[dst_v.at[j]], add=True)

    plsc.subcore_barrier()
    pltpu.sync_copy(deg_sh.at[pl.ds(nbase, ROWS_PER_TILE)], out_hbm.at[wid])


def _scale_body(deg_ref, x_ref, dis_ref, y_ref):
    d = deg_ref[0, 0, :, 0:1] + deg_ref[0, 1, :, 0:1] + 1.0
    di = lax.rsqrt(d)
    dis_ref[...] = di
    y_ref[...] = x_ref[...] * di


def _tc_scale(deg4, x_pad):
    return pl.pallas_call(
        _scale_body,
        grid=(N_PAD // 128,),
        in_specs=[
            pl.BlockSpec((1, NC, 128, DEG_W), lambda i: (i // 5, 0, i % 5, 0)),
            pl.BlockSpec((128, D_IN), lambda i: (i, 0)),
        ],
        out_specs=[
            pl.BlockSpec((128, 1), lambda i: (i, 0)),
            pl.BlockSpec((128, D_IN), lambda i: (i, 0)),
        ],
        out_shape=[
            jax.ShapeDtypeStruct((N_PAD, 1), jnp.float32),
            jax.ShapeDtypeStruct((N_PAD, D_IN), jnp.float32),
        ],
    )(deg4, x_pad)


GCH = 128
NCHUNKS = E_PAD // GCH
NBUF = 2
GIB = 16
G0 = 112
G1 = NCHUNKS // NS - G0


@functools.partial(
    pl.kernel,
    out_type=jax.ShapeDtypeStruct((NW, ROWS_PER_TILE, D_IN), jnp.float32),
    mesh=_mesh,
    scratch_types=[
        pltpu.VMEM((GIB, GCH), jnp.int32),
        pltpu.VMEM((GIB, GCH), jnp.int32),
        [pltpu.VMEM((GCH, D_IN), jnp.float32) for _ in range(NBUF)],
        pltpu.VMEM_SHARED((N_PAD, D_IN), jnp.float32),
        [pltpu.SemaphoreType.DMA for _ in range(NBUF)],
    ],
)
def _sc_aggregate(src_hbm, dst_hbm, y_hbm, zeros_hbm, out_hbm,
                  src_b, dst_b, bufs, s_sh, sems):
    cid = lax.axis_index("c")
    sid = lax.axis_index("s")
    wid = sid * NC + cid
    ebase = jnp.where(cid == 0, sid * G0, NS * G0 + sid * G1)
    nrefill = jnp.where(cid == 0, G0 // GIB, G1 // GIB)
    nbase = sid * ROWS_PER_TILE
    with jax.named_scope("agg_init"):
        pltpu.sync_copy(zeros_hbm.at[pl.ds(nbase, ROWS_PER_TILE)],
                        s_sh.at[pl.ds(nbase, ROWS_PER_TILE)])
        plsc.subcore_barrier()

    with jax.named_scope("agg_edges"):
        @pl.loop(0, nrefill)
        def _(b):
            cbase = ebase + b * GIB
            pltpu.sync_copy(src_hbm.at[pl.ds(cbase, GIB)], src_b)
            pltpu.sync_copy(dst_hbm.at[pl.ds(cbase, GIB)], dst_b)
            for k in range(NBUF - 1):
                pltpu.async_copy(y_hbm.at[src_b.at[k]], bufs[k], sems[k])

            @pl.loop(0, GIB // NBUF)
            def _(t):
                for k in range(NBUF):
                    j = NBUF * t + k
                    jn = j + NBUF - 1
                    kn = (k + NBUF - 1) % NBUF

                    @pl.when(jn < GIB)
                    def _():
                        pltpu.async_copy(y_hbm.at[src_b.at[jn]], bufs[kn],
                                         sems[kn])

                    pltpu.make_async_copy(y_hbm.at[src_b.at[j]], bufs[k],
                                          sems[k]).wait()
                    pltpu.sync_copy(bufs[k], s_sh.at[dst_b.at[j]], add=True)

    with jax.named_scope("agg_drain"):
        plsc.subcore_barrier()
        pltpu.sync_copy(s_sh.at[pl.ds(nbase, ROWS_PER_TILE)], out_hbm.at[wid])


def _mlp_body(s_ref, y_ref, dis_ref, wc_ref, bc_ref, w1_ref, b1_ref,
              w2_ref, b2_ref, out_ref):
    z = (s_ref[0, 0] + s_ref[0, 1] + y_ref[...]) * dis_ref[...]
    h = jnp.dot(z, wc_ref[...], preferred_element_type=jnp.float32,
                precision=lax.Precision.HIGHEST) + bc_ref[...]
    h = jnp.maximum(h, 0.0)
    h = jnp.dot(h, w1_ref[...], preferred_element_type=jnp.float32,
                precision=lax.Precision.HIGHEST) + b1_ref[...]
    h = jnp.maximum(h, 0.0)
    out_ref[...] = jnp.dot(h, w2_ref[...], preferred_element_type=jnp.float32,
                           precision=lax.Precision.HIGHEST) + b2_ref[...]


def _tc_mlp(s4, y, dis, W_conv, b_conv, W_fc1, b_fc1, W_fc2, b_fc2):
    full = lambda shape: pl.BlockSpec(shape, lambda i: tuple(0 for _ in shape))
    return pl.pallas_call(
        _mlp_body,
        grid=(NS,),
        in_specs=[
            pl.BlockSpec((1, NC, ROWS_PER_TILE, D_IN), lambda i: (i, 0, 0, 0)),
            pl.BlockSpec((ROWS_PER_TILE, D_IN), lambda i: (i, 0)),
            pl.BlockSpec((ROWS_PER_TILE, 1), lambda i: (i, 0)),
            full((D_IN, D1)),
            full((1, D1)),
            full((D1, D2)),
            full((1, D2)),
            full((D2, D_IN)),
            full((1, D_IN)),
        ],
        out_specs=pl.BlockSpec((ROWS_PER_TILE, D_IN), lambda i: (i, 0)),
        out_shape=jax.ShapeDtypeStruct((N_PAD, D_IN), jnp.float32),
    )(s4, y, dis, W_conv, b_conv, W_fc1, b_fc1, W_fc2, b_fc2)


def kernel(x, edge_index, W_conv, b_conv, W_fc1, b_fc1, W_fc2, b_fc2):
    ei = edge_index.astype(jnp.int32)
    pad_e = E_PAD - E
    src = jnp.concatenate([ei[0], jnp.full((pad_e,), N, jnp.int32)])
    dst = jnp.concatenate([ei[1], jnp.full((pad_e,), N_PAD - 1, jnp.int32)])
    dst2d = dst.reshape(-1, CHUNK)
    src2g = src.reshape(-1, GCH)
    dst2g = dst.reshape(-1, GCH)
    x_pad = jnp.concatenate(
        [x, jnp.zeros((N_PAD - N, D_IN), jnp.float32)], axis=0)

    zeros_d = jnp.zeros((N_PAD, DEG_W), jnp.float32)
    ones_w = jnp.ones((CHUNK, DEG_W), jnp.float32)
    deg_parts = _sc_degree(dst2d, zeros_d, ones_w)
    deg4 = deg_parts.reshape(NS, NC, ROWS_PER_TILE, DEG_W)

    dis, y = _tc_scale(deg4, x_pad)

    zeros_s = jnp.zeros((N_PAD, D_IN), jnp.float32)
    s_parts = _sc_aggregate(src2g, dst2g, y, zeros_s)
    s4 = s_parts.reshape(NS, NC, ROWS_PER_TILE, D_IN)

    out = _tc_mlp(s4, y, dis, W_conv, b_conv.reshape(1, D1),
                  W_fc1, b_fc1.reshape(1, D2), W_fc2, b_fc2.reshape(1, D_IN))
    return out[:N]


# speedup vs baseline: 1.8850x; 1.8850x over previous
import functools

import jax
import jax.numpy as jnp
from jax import lax
from jax.experimental import pallas as pl
from jax.experimental.pallas import tpu as pltpu
from jax.experimental.pallas import tpu_sc as plsc

N = 10000
E = 320000
D_IN = 128
D1 = 256
D2 = 128

NC = 2
NS = 16
NW = NC * NS

N_PAD = 10240
ROWS_PER_TILE = N_PAD // NS
CHUNK = 128
E_PAD = 327680
CHUNKS_PER_TILE = E_PAD // (NW * CHUNK)
DEG_W = 128

_mesh = plsc.VectorSubcoreMesh(
    core_axis_name="c", subcore_axis_name="s", num_cores=NC, num_subcores=NS)


@functools.partial(
    pl.kernel,
    out_type=jax.ShapeDtypeStruct((NW, ROWS_PER_TILE, DEG_W), jnp.float32),
    mesh=_mesh,
    scratch_types=[
        pltpu.VMEM((CHUNKS_PER_TILE, CHUNK), jnp.int32),
        pltpu.VMEM((CHUNK, DEG_W), jnp.float32),
        pltpu.VMEM_SHARED((N_PAD, DEG_W), jnp.float32),
    ],
)
def _sc_degree(dst_hbm, zeros_hbm, ones_hbm, out_hbm, dst_v, ones_v, deg_sh):
    cid = lax.axis_index("c")
    sid = lax.axis_index("s")
    wid = sid * NC + cid
    nbase = sid * ROWS_PER_TILE
    pltpu.sync_copy(dst_hbm.at[pl.ds(wid * CHUNKS_PER_TILE, CHUNKS_PER_TILE)],
                    dst_v)
    pltpu.sync_copy(ones_hbm, ones_v)
    pltpu.sync_copy(zeros_hbm.at[pl.ds(nbase, ROWS_PER_TILE)],
                    deg_sh.at[pl.ds(nbase, ROWS_PER_TILE)])
    plsc.subcore_barrier()

    @pl.loop(0, CHUNKS_PER_TILE)
    def _(j):
        pltpu.sync_copy(ones_v, deg_sh.at[dst_v.at[j]], add=True)

    plsc.subcore_barrier()
    pltpu.sync_copy(deg_sh.at[pl.ds(nbase, ROWS_PER_TILE)], out_hbm.at[wid])


def _scale_body(deg_ref, x_ref, dis_ref, y_ref):
    d = deg_ref[0, 0, :, 0:1] + deg_ref[0, 1, :, 0:1] + 1.0
    di = lax.rsqrt(d)
    dis_ref[...] = di
    y_ref[...] = x_ref[...] * di


def _tc_scale(deg4, x_pad):
    return pl.pallas_call(
        _scale_body,
        grid=(N_PAD // 128,),
        in_specs=[
            pl.BlockSpec((1, NC, 128, DEG_W), lambda i: (i // 5, 0, i % 5, 0)),
            pl.BlockSpec((128, D_IN), lambda i: (i, 0)),
        ],
        out_specs=[
            pl.BlockSpec((128, 1), lambda i: (i, 0)),
            pl.BlockSpec((128, D_IN), lambda i: (i, 0)),
        ],
        out_shape=[
            jax.ShapeDtypeStruct((N_PAD, 1), jnp.float32),
            jax.ShapeDtypeStruct((N_PAD, D_IN), jnp.float32),
        ],
    )(deg4, x_pad)


GCH = 128
NCHUNKS = E_PAD // GCH
NBUF = 2
GIB = 16
G0 = 80
G1 = NCHUNKS // NS - G0


@functools.partial(
    pl.kernel,
    out_type=jax.ShapeDtypeStruct((NW, ROWS_PER_TILE, D_IN), jnp.float32),
    mesh=_mesh,
    scratch_types=[
        pltpu.VMEM((GIB, GCH), jnp.int32),
        pltpu.VMEM((GIB, GCH), jnp.int32),
        [pltpu.VMEM((GCH, D_IN), jnp.float32) for _ in range(NBUF)],
        pltpu.VMEM_SHARED((N_PAD, D_IN), jnp.float32),
        [pltpu.SemaphoreType.DMA for _ in range(NBUF)],
    ],
)
def _sc_aggregate(src_hbm, dst_hbm, y_hbm, zeros_hbm, out_hbm,
                  src_b, dst_b, bufs, s_sh, sems):
    cid = lax.axis_index("c")
    sid = lax.axis_index("s")
    wid = sid * NC + cid
    ebase = jnp.where(cid == 0, sid * G0, NS * G0 + sid * G1)
    nrefill = jnp.where(cid == 0, G0 // GIB, G1 // GIB)
    nbase = sid * ROWS_PER_TILE
    with jax.named_scope("agg_init"):
        pltpu.sync_copy(zeros_hbm.at[pl.ds(nbase, ROWS_PER_TILE)],
                        s_sh.at[pl.ds(nbase, ROWS_PER_TILE)])
        plsc.subcore_barrier()

    with jax.named_scope("agg_edges"):
        @pl.loop(0, nrefill)
        def _(b):
            cbase = ebase + b * GIB
            pltpu.sync_copy(src_hbm.at[pl.ds(cbase, GIB)], src_b)
            pltpu.sync_copy(dst_hbm.at[pl.ds(cbase, GIB)], dst_b)
            for k in range(NBUF - 1):
                pltpu.async_copy(y_hbm.at[src_b.at[k]], bufs[k], sems[k])

            @pl.loop(0, GIB // NBUF)
            def _(t):
                for k in range(NBUF):
                    j = NBUF * t + k
                    jn = j + NBUF - 1
                    kn = (k + NBUF - 1) % NBUF

                    @pl.when(jn < GIB)
                    def _():
                        pltpu.async_copy(y_hbm.at[src_b.at[jn]], bufs[kn],
                                         sems[kn])

                    pltpu.make_async_copy(y_hbm.at[src_b.at[j]], bufs[k],
                                          sems[k]).wait()
                    pltpu.sync_copy(bufs[k], s_sh.at[dst_b.at[j]], add=True)

    with jax.named_scope("agg_drain"):
        plsc.subcore_barrier()
        pltpu.sync_copy(s_sh.at[pl.ds(nbase, ROWS_PER_TILE)], out_hbm.at[wid])


def _mlp_body(s_ref, y_ref, dis_ref, wc_ref, bc_ref, w1_ref, b1_ref,
              w2_ref, b2_ref, out_ref):
    z = (s_ref[0, 0] + s_ref[0, 1] + y_ref[...]) * dis_ref[...]
    h = jnp.dot(z, wc_ref[...], preferred_element_type=jnp.float32,
                precision=lax.Precision.HIGHEST) + bc_ref[...]
    h = jnp.maximum(h, 0.0)
    h = jnp.dot(h, w1_ref[...], preferred_element_type=jnp.float32,
                precision=lax.Precision.HIGHEST) + b1_ref[...]
    h = jnp.maximum(h, 0.0)
    out_ref[...] = jnp.dot(h, w2_ref[...], preferred_element_type=jnp.float32,
                           precision=lax.Precision.HIGHEST) + b2_ref[...]


def _tc_mlp(s4, y, dis, W_conv, b_conv, W_fc1, b_fc1, W_fc2, b_fc2):
    full = lambda shape: pl.BlockSpec(shape, lambda i: tuple(0 for _ in shape))
    return pl.pallas_call(
        _mlp_body,
        grid=(NS,),
        in_specs=[
            pl.BlockSpec((1, NC, ROWS_PER_TILE, D_IN), lambda i: (i, 0, 0, 0)),
            pl.BlockSpec((ROWS_PER_TILE, D_IN), lambda i: (i, 0)),
            pl.BlockSpec((ROWS_PER_TILE, 1), lambda i: (i, 0)),
            full((D_IN, D1)),
            full((1, D1)),
            full((D1, D2)),
            full((1, D2)),
            full((D2, D_IN)),
            full((1, D_IN)),
        ],
        out_specs=pl.BlockSpec((ROWS_PER_TILE, D_IN), lambda i: (i, 0)),
        out_shape=jax.ShapeDtypeStruct((N_PAD, D_IN), jnp.float32),
    )(s4, y, dis, W_conv, b_conv, W_fc1, b_fc1, W_fc2, b_fc2)


def kernel(x, edge_index, W_conv, b_conv, W_fc1, b_fc1, W_fc2, b_fc2):
    ei = edge_index.astype(jnp.int32)
    pad_e = E_PAD - E
    src = jnp.concatenate(
        [ei[0], (jnp.arange(pad_e, dtype=jnp.int32) * 32) % N])
    dst = jnp.concatenate([ei[1], jnp.full((pad_e,), N_PAD - 1, jnp.int32)])
    dst2d = dst.reshape(-1, CHUNK)
    src2g = src.reshape(-1, GCH)
    dst2g = dst.reshape(-1, GCH)
    x_pad = jnp.concatenate(
        [x, jnp.zeros((N_PAD - N, D_IN), jnp.float32)], axis=0)

    zeros_d = jnp.zeros((N_PAD, DEG_W), jnp.float32)
    ones_w = jnp.ones((CHUNK, DEG_W), jnp.float32)
    deg_parts = _sc_degree(dst2d, zeros_d, ones_w)
    deg4 = deg_parts.reshape(NS, NC, ROWS_PER_TILE, DEG_W)

    dis, y = _tc_scale(deg4, x_pad)

    zeros_s = jnp.zeros((N_PAD, D_IN), jnp.float32)
    s_parts = _sc_aggregate(src2g, dst2g, y, zeros_s)
    s4 = s_parts.reshape(NS, NC, ROWS_PER_TILE, D_IN)

    out = _tc_mlp(s4, y, dis, W_conv, b_conv.reshape(1, D1),
                  W_fc1, b_fc1.reshape(1, D2), W_fc2, b_fc2.reshape(1, D_IN))
    return out[:N]


# speedup vs baseline: 2.2844x; 1.2119x over previous
import functools

import jax
import jax.numpy as jnp
from jax import lax
from jax.experimental import pallas as pl
from jax.experimental.pallas import tpu as pltpu
from jax.experimental.pallas import tpu_sc as plsc

N = 10000
E = 320000
D_IN = 128
D1 = 256
D2 = 128

NC = 2
NS = 16
NW = NC * NS

N_PAD = 10240
ROWS_PER_TILE = N_PAD // NS
CHUNK = 128
E_PAD = 327680
CHUNKS_PER_TILE = E_PAD // (NW * CHUNK)
DEG_W = 128

_mesh = plsc.VectorSubcoreMesh(
    core_axis_name="c", subcore_axis_name="s", num_cores=NC, num_subcores=NS)


@functools.partial(
    pl.kernel,
    out_type=jax.ShapeDtypeStruct((NW, ROWS_PER_TILE, DEG_W), jnp.float32),
    mesh=_mesh,
    scratch_types=[
        pltpu.VMEM((CHUNKS_PER_TILE, CHUNK), jnp.int32),
        pltpu.VMEM((CHUNK, DEG_W), jnp.float32),
        pltpu.VMEM_SHARED((N_PAD, DEG_W), jnp.float32),
    ],
)
def _sc_degree(dst_hbm, zeros_hbm, ones_hbm, out_hbm, dst_v, ones_v, deg_sh):
    cid = lax.axis_index("c")
    sid = lax.axis_index("s")
    wid = sid * NC + cid
    nbase = sid * ROWS_PER_TILE
    pltpu.sync_copy(dst_hbm.at[pl.ds(wid * CHUNKS_PER_TILE, CHUNKS_PER_TILE)],
                    dst_v)
    pltpu.sync_copy(ones_hbm, ones_v)
    pltpu.sync_copy(zeros_hbm.at[pl.ds(nbase, ROWS_PER_TILE)],
                    deg_sh.at[pl.ds(nbase, ROWS_PER_TILE)])
    plsc.subcore_barrier()

    @pl.loop(0, CHUNKS_PER_TILE)
    def _(j):
        pltpu.sync_copy(ones_v, deg_sh.at[dst_v.at[j]], add=True)

    plsc.subcore_barrier()
    pltpu.sync_copy(deg_sh.at[pl.ds(nbase, ROWS_PER_TILE)], out_hbm.at[wid])


def _scale_body(deg_ref, x_ref, dis_ref, y_ref):
    d = deg_ref[0, 0, :, 0:1] + deg_ref[0, 1, :, 0:1] + 1.0
    di = lax.rsqrt(d)
    dis_ref[...] = di
    y_ref[...] = x_ref[...] * di


def _tc_scale(deg4, x_pad):
    return pl.pallas_call(
        _scale_body,
        grid=(NS,),
        in_specs=[
            pl.BlockSpec((1, NC, ROWS_PER_TILE, DEG_W), lambda i: (i, 0, 0, 0)),
            pl.BlockSpec((ROWS_PER_TILE, D_IN), lambda i: (i, 0)),
        ],
        out_specs=[
            pl.BlockSpec((ROWS_PER_TILE, 1), lambda i: (i, 0)),
            pl.BlockSpec((ROWS_PER_TILE, D_IN), lambda i: (i, 0)),
        ],
        out_shape=[
            jax.ShapeDtypeStruct((N_PAD, 1), jnp.float32),
            jax.ShapeDtypeStruct((N_PAD, D_IN), jnp.float32),
        ],
    )(deg4, x_pad)


GCH = 128
NCHUNKS = E_PAD // GCH
NBUF = 2
GIB = 16
G0 = 80
G1 = NCHUNKS // NS - G0


@functools.partial(
    pl.kernel,
    out_type=jax.ShapeDtypeStruct((NW, ROWS_PER_TILE, D_IN), jnp.float32),
    mesh=_mesh,
    scratch_types=[
        pltpu.VMEM((GIB, GCH), jnp.int32),
        pltpu.VMEM((GIB, GCH), jnp.int32),
        [pltpu.VMEM((GCH, D_IN), jnp.float32) for _ in range(NBUF)],
        pltpu.VMEM_SHARED((N_PAD, D_IN), jnp.float32),
        [pltpu.SemaphoreType.DMA for _ in range(NBUF)],
    ],
)
def _sc_aggregate(src_hbm, dst_hbm, y_hbm, zeros_hbm, out_hbm,
                  src_b, dst_b, bufs, s_sh, sems):
    cid = lax.axis_index("c")
    sid = lax.axis_index("s")
    wid = sid * NC + cid
    ebase = jnp.where(cid == 0, sid * G0, NS * G0 + sid * G1)
    nrefill = jnp.where(cid == 0, G0 // GIB, G1 // GIB)
    nbase = sid * ROWS_PER_TILE
    with jax.named_scope("agg_init"):
        pltpu.sync_copy(zeros_hbm.at[pl.ds(nbase, ROWS_PER_TILE)],
                        s_sh.at[pl.ds(nbase, ROWS_PER_TILE)])
        plsc.subcore_barrier()

    with jax.named_scope("agg_edges"):
        @pl.loop(0, nrefill)
        def _(b):
            cbase = ebase + b * GIB
            pltpu.sync_copy(src_hbm.at[pl.ds(cbase, GIB)], src_b)
            pltpu.sync_copy(dst_hbm.at[pl.ds(cbase, GIB)], dst_b)
            for k in range(NBUF - 1):
                pltpu.async_copy(y_hbm.at[src_b.at[k]], bufs[k], sems[k])

            @pl.loop(0, GIB // NBUF)
            def _(t):
                for k in range(NBUF):
                    j = NBUF * t + k
                    jn = j + NBUF - 1
                    kn = (k + NBUF - 1) % NBUF

                    @pl.when(jn < GIB)
                    def _():
                        pltpu.async_copy(y_hbm.at[src_b.at[jn]], bufs[kn],
                                         sems[kn])

                    pltpu.make_async_copy(y_hbm.at[src_b.at[j]], bufs[k],
                                          sems[k]).wait()
                    pltpu.sync_copy(bufs[k], s_sh.at[dst_b.at[j]], add=True)

    with jax.named_scope("agg_drain"):
        plsc.subcore_barrier()
        pltpu.sync_copy(s_sh.at[pl.ds(nbase, ROWS_PER_TILE)], out_hbm.at[wid])


def _mlp_body(s_ref, y_ref, dis_ref, wc_ref, bc_ref, w1_ref, b1_ref,
              w2_ref, b2_ref, out_ref):
    z = (s_ref[0, 0] + s_ref[0, 1] + y_ref[...]) * dis_ref[...]
    h = jnp.dot(z, wc_ref[...], preferred_element_type=jnp.float32,
                precision=lax.Precision.DEFAULT) + bc_ref[...]
    h = jnp.maximum(h, 0.0)
    h = jnp.dot(h, w1_ref[...], preferred_element_type=jnp.float32,
                precision=lax.Precision.DEFAULT) + b1_ref[...]
    h = jnp.maximum(h, 0.0)
    out_ref[...] = jnp.dot(h, w2_ref[...], preferred_element_type=jnp.float32,
                           precision=lax.Precision.DEFAULT) + b2_ref[...]


def _tc_mlp(s4, y, dis, W_conv, b_conv, W_fc1, b_fc1, W_fc2, b_fc2):
    full = lambda shape: pl.BlockSpec(shape, lambda i: tuple(0 for _ in shape))
    return pl.pallas_call(
        _mlp_body,
        grid=(NS,),
        in_specs=[
            pl.BlockSpec((1, NC, ROWS_PER_TILE, D_IN), lambda i: (i, 0, 0, 0)),
            pl.BlockSpec((ROWS_PER_TILE, D_IN), lambda i: (i, 0)),
            pl.BlockSpec((ROWS_PER_TILE, 1), lambda i: (i, 0)),
            full((D_IN, D1)),
            full((1, D1)),
            full((D1, D2)),
            full((1, D2)),
            full((D2, D_IN)),
            full((1, D_IN)),
        ],
        out_specs=pl.BlockSpec((ROWS_PER_TILE, D_IN), lambda i: (i, 0)),
        out_shape=jax.ShapeDtypeStruct((N_PAD, D_IN), jnp.float32),
    )(s4, y, dis, W_conv, b_conv, W_fc1, b_fc1, W_fc2, b_fc2)


def kernel(x, edge_index, W_conv, b_conv, W_fc1, b_fc1, W_fc2, b_fc2):
    ei = edge_index.astype(jnp.int32)
    pad_e = E_PAD - E
    src = jnp.concatenate(
        [ei[0], (jnp.arange(pad_e, dtype=jnp.int32) * 32) % N])
    dst = jnp.concatenate([ei[1], jnp.full((pad_e,), N_PAD - 1, jnp.int32)])
    dst2d = dst.reshape(-1, CHUNK)
    src2g = src.reshape(-1, GCH)
    dst2g = dst.reshape(-1, GCH)
    x_pad = jnp.concatenate(
        [x, jnp.zeros((N_PAD - N, D_IN), jnp.float32)], axis=0)

    zeros_d = jnp.zeros((N_PAD, DEG_W), jnp.float32)
    ones_w = jnp.ones((CHUNK, DEG_W), jnp.float32)
    deg_parts = _sc_degree(dst2d, zeros_d, ones_w)
    deg4 = deg_parts.reshape(NS, NC, ROWS_PER_TILE, DEG_W)

    dis, y = _tc_scale(deg4, x_pad)

    zeros_s = jnp.zeros((N_PAD, D_IN), jnp.float32)
    s_parts = _sc_aggregate(src2g, dst2g, y, zeros_s)
    s4 = s_parts.reshape(NS, NC, ROWS_PER_TILE, D_IN)

    out = _tc_mlp(s4, y, dis, W_conv, b_conv.reshape(1, D1),
                  W_fc1, b_fc1.reshape(1, D2), W_fc2, b_fc2.reshape(1, D_IN))
    return out[:N]


# speedup vs baseline: 2.3245x; 1.0175x over previous
import functools

import jax
import jax.numpy as jnp
from jax import lax
from jax.experimental import pallas as pl
from jax.experimental.pallas import tpu as pltpu
from jax.experimental.pallas import tpu_sc as plsc

N = 10000
E = 320000
D_IN = 128
D1 = 256
D2 = 128

NC = 2
NS = 16
NW = NC * NS

N_PAD = 10240
ROWS_PER_TILE = N_PAD // NS
CHUNK = 128
E_PAD = 327680
CHUNKS_PER_TILE = E_PAD // (NW * CHUNK)
DEG_W = 128

_mesh = plsc.VectorSubcoreMesh(
    core_axis_name="c", subcore_axis_name="s", num_cores=NC, num_subcores=NS)


@functools.partial(
    pl.kernel,
    out_type=jax.ShapeDtypeStruct((NW, ROWS_PER_TILE, DEG_W), jnp.float32),
    mesh=_mesh,
    scratch_types=[
        pltpu.VMEM((CHUNKS_PER_TILE, CHUNK), jnp.int32),
        pltpu.VMEM((CHUNK, DEG_W), jnp.float32),
        pltpu.VMEM_SHARED((N_PAD, DEG_W), jnp.float32),
        pltpu.SemaphoreType.DMA,
    ],
)
def _sc_degree(dst_hbm, zeros_hbm, ones_hbm, out_hbm, dst_v, ones_v, deg_sh,
               ssem):
    cid = lax.axis_index("c")
    sid = lax.axis_index("s")
    wid = sid * NC + cid
    nbase = sid * ROWS_PER_TILE
    pltpu.sync_copy(dst_hbm.at[pl.ds(wid * CHUNKS_PER_TILE, CHUNKS_PER_TILE)],
                    dst_v)
    pltpu.sync_copy(ones_hbm, ones_v)
    pltpu.sync_copy(zeros_hbm.at[pl.ds(nbase, ROWS_PER_TILE)],
                    deg_sh.at[pl.ds(nbase, ROWS_PER_TILE)])
    plsc.subcore_barrier()

    @pl.loop(0, CHUNKS_PER_TILE)
    def _(j):
        pltpu.async_copy(ones_v, deg_sh.at[dst_v.at[j]], ssem, add=True)

    @pl.loop(0, CHUNKS_PER_TILE)
    def _(j):
        pltpu.make_async_copy(ones_v, deg_sh.at[dst_v.at[j]], ssem).wait()

    plsc.subcore_barrier()
    pltpu.sync_copy(deg_sh.at[pl.ds(nbase, ROWS_PER_TILE)], out_hbm.at[wid])


def _scale_body(deg_ref, x_ref, dis_ref, y_ref):
    d = deg_ref[0, 0, :, 0:1] + deg_ref[0, 1, :, 0:1] + 1.0
    di = lax.rsqrt(d)
    dis_ref[...] = di
    y_ref[...] = x_ref[...] * di


def _tc_scale(deg4, x_pad):
    return pl.pallas_call(
        _scale_body,
        grid=(NS,),
        in_specs=[
            pl.BlockSpec((1, NC, ROWS_PER_TILE, DEG_W), lambda i: (i, 0, 0, 0)),
            pl.BlockSpec((ROWS_PER_TILE, D_IN), lambda i: (i, 0)),
        ],
        out_specs=[
            pl.BlockSpec((ROWS_PER_TILE, 1), lambda i: (i, 0)),
            pl.BlockSpec((ROWS_PER_TILE, D_IN), lambda i: (i, 0)),
        ],
        out_shape=[
            jax.ShapeDtypeStruct((N_PAD, 1), jnp.float32),
            jax.ShapeDtypeStruct((N_PAD, D_IN), jnp.float32),
        ],
    )(deg4, x_pad)


GCH = 128
NCHUNKS = E_PAD // GCH
NBUF = 2
GIB = 16
G0 = 80
G1 = NCHUNKS // NS - G0


@functools.partial(
    pl.kernel,
    out_type=jax.ShapeDtypeStruct((NW, ROWS_PER_TILE, D_IN), jnp.float32),
    mesh=_mesh,
    scratch_types=[
        pltpu.VMEM((GIB, GCH), jnp.int32),
        pltpu.VMEM((GIB, GCH), jnp.int32),
        [pltpu.VMEM((GCH, D_IN), jnp.float32) for _ in range(NBUF)],
        pltpu.VMEM_SHARED((N_PAD, D_IN), jnp.float32),
        [pltpu.SemaphoreType.DMA for _ in range(NBUF)],
        [pltpu.SemaphoreType.DMA for _ in range(NBUF)],
    ],
)
def _sc_aggregate(src_hbm, dst_hbm, y_hbm, zeros_hbm, out_hbm,
                  src_b, dst_b, bufs, s_sh, sems, ssems):
    cid = lax.axis_index("c")
    sid = lax.axis_index("s")
    wid = sid * NC + cid
    ebase = jnp.where(cid == 0, sid * G0, NS * G0 + sid * G1)
    nrefill = jnp.where(cid == 0, G0 // GIB, G1 // GIB)
    nbase = sid * ROWS_PER_TILE
    with jax.named_scope("agg_init"):
        pltpu.sync_copy(zeros_hbm.at[pl.ds(nbase, ROWS_PER_TILE)],
                        s_sh.at[pl.ds(nbase, ROWS_PER_TILE)])
        plsc.subcore_barrier()

    with jax.named_scope("agg_edges"):
        @pl.loop(0, nrefill)
        def _(b):
            cbase = ebase + b * GIB
            pltpu.sync_copy(src_hbm.at[pl.ds(cbase, GIB)], src_b)
            pltpu.sync_copy(dst_hbm.at[pl.ds(cbase, GIB)], dst_b)
            for k in range(NBUF - 1):
                pltpu.async_copy(y_hbm.at[src_b.at[k]], bufs[k], sems[k])

            @pl.loop(0, GIB // NBUF)
            def _(t):
                for k in range(NBUF):
                    j = NBUF * t + k
                    jn = j + NBUF - 1
                    kn = (k + NBUF - 1) % NBUF

                    @pl.when(jn < GIB)
                    def _():
                        @pl.when(jn >= NBUF)
                        def _():
                            pltpu.make_async_copy(
                                bufs[kn], s_sh.at[dst_b.at[jn - NBUF]],
                                ssems[kn]).wait()

                        pltpu.async_copy(y_hbm.at[src_b.at[jn]], bufs[kn],
                                         sems[kn])

                    pltpu.make_async_copy(y_hbm.at[src_b.at[j]], bufs[k],
                                          sems[k]).wait()
                    pltpu.async_copy(bufs[k], s_sh.at[dst_b.at[j]], ssems[k],
                                     add=True)

            for k in range(NBUF):
                pltpu.make_async_copy(bufs[k], s_sh.at[dst_b.at[k]],
                                      ssems[k]).wait()

    with jax.named_scope("agg_drain"):
        plsc.subcore_barrier()
        pltpu.sync_copy(s_sh.at[pl.ds(nbase, ROWS_PER_TILE)], out_hbm.at[wid])


def _mlp_body(s_ref, y_ref, dis_ref, wc_ref, bc_ref, w1_ref, b1_ref,
              w2_ref, b2_ref, out_ref):
    z = (s_ref[0, 0] + s_ref[0, 1] + y_ref[...]) * dis_ref[...]
    h = jnp.dot(z, wc_ref[...], preferred_element_type=jnp.float32,
                precision=lax.Precision.DEFAULT) + bc_ref[...]
    h = jnp.maximum(h, 0.0)
    h = jnp.dot(h, w1_ref[...], preferred_element_type=jnp.float32,
                precision=lax.Precision.DEFAULT) + b1_ref[...]
    h = jnp.maximum(h, 0.0)
    out_ref[...] = jnp.dot(h, w2_ref[...], preferred_element_type=jnp.float32,
                           precision=lax.Precision.DEFAULT) + b2_ref[...]


def _tc_mlp(s4, y, dis, W_conv, b_conv, W_fc1, b_fc1, W_fc2, b_fc2):
    full = lambda shape: pl.BlockSpec(shape, lambda i: tuple(0 for _ in shape))
    return pl.pallas_call(
        _mlp_body,
        grid=(NS,),
        in_specs=[
            pl.BlockSpec((1, NC, ROWS_PER_TILE, D_IN), lambda i: (i, 0, 0, 0)),
            pl.BlockSpec((ROWS_PER_TILE, D_IN), lambda i: (i, 0)),
            pl.BlockSpec((ROWS_PER_TILE, 1), lambda i: (i, 0)),
            full((D_IN, D1)),
            full((1, D1)),
            full((D1, D2)),
            full((1, D2)),
            full((D2, D_IN)),
            full((1, D_IN)),
        ],
        out_specs=pl.BlockSpec((ROWS_PER_TILE, D_IN), lambda i: (i, 0)),
        out_shape=jax.ShapeDtypeStruct((N, D_IN), jnp.float32),
    )(s4, y, dis, W_conv, b_conv, W_fc1, b_fc1, W_fc2, b_fc2)


def kernel(x, edge_index, W_conv, b_conv, W_fc1, b_fc1, W_fc2, b_fc2):
    ei = edge_index.astype(jnp.int32)
    pad_e = E_PAD - E
    src = jnp.concatenate(
        [ei[0], (jnp.arange(pad_e, dtype=jnp.int32) * 32) % N])
    dst = jnp.concatenate([ei[1], jnp.full((pad_e,), N_PAD - 1, jnp.int32)])
    dst2d = dst.reshape(-1, CHUNK)
    src2g = src.reshape(-1, GCH)
    dst2g = dst.reshape(-1, GCH)
    x_pad = jnp.concatenate(
        [x, jnp.zeros((N_PAD - N, D_IN), jnp.float32)], axis=0)

    zeros_d = jnp.zeros((N_PAD, DEG_W), jnp.float32)
    ones_w = jnp.ones((CHUNK, DEG_W), jnp.float32)
    deg_parts = _sc_degree(dst2d, zeros_d, ones_w)
    deg4 = deg_parts.reshape(NS, NC, ROWS_PER_TILE, DEG_W)

    dis, y = _tc_scale(deg4, x_pad)

    zeros_s = jnp.zeros((N_PAD, D_IN), jnp.float32)
    s_parts = _sc_aggregate(src2g, dst2g, y, zeros_s)
    s4 = s_parts.reshape(NS, NC, ROWS_PER_TILE, D_IN)

    return _tc_mlp(s4, y, dis, W_conv, b_conv.reshape(1, D1),
                   W_fc1, b_fc1.reshape(1, D2), W_fc2, b_fc2.reshape(1, D_IN))


# speedup vs baseline: 2.3270x; 1.0011x over previous
import functools

import jax
import jax.numpy as jnp
from jax import lax
from jax.experimental import pallas as pl
from jax.experimental.pallas import tpu as pltpu
from jax.experimental.pallas import tpu_sc as plsc

N = 10000
E = 320000
D_IN = 128
D1 = 256
D2 = 128

NC = 2
NS = 16
NW = NC * NS

N_PAD = 10240
ROWS_PER_TILE = N_PAD // NS
CHUNK = 128
E_PAD = 327680
CHUNKS_PER_TILE = E_PAD // (NW * CHUNK)
DEG_W = 128

_mesh = plsc.VectorSubcoreMesh(
    core_axis_name="c", subcore_axis_name="s", num_cores=NC, num_subcores=NS)


@functools.partial(
    pl.kernel,
    out_type=jax.ShapeDtypeStruct((NW, ROWS_PER_TILE, DEG_W), jnp.float32),
    mesh=_mesh,
    scratch_types=[
        pltpu.VMEM((CHUNKS_PER_TILE, CHUNK), jnp.int32),
        pltpu.VMEM((CHUNK, DEG_W), jnp.float32),
        pltpu.VMEM_SHARED((N_PAD, DEG_W), jnp.float32),
        pltpu.SemaphoreType.DMA,
    ],
)
def _sc_degree(dst_hbm, zeros_hbm, ones_hbm, out_hbm, dst_v, ones_v, deg_sh,
               ssem):
    cid = lax.axis_index("c")
    sid = lax.axis_index("s")
    wid = sid * NC + cid
    nbase = sid * ROWS_PER_TILE
    pltpu.sync_copy(dst_hbm.at[pl.ds(wid * CHUNKS_PER_TILE, CHUNKS_PER_TILE)],
                    dst_v)
    pltpu.sync_copy(ones_hbm, ones_v)
    pltpu.sync_copy(zeros_hbm.at[pl.ds(nbase, ROWS_PER_TILE)],
                    deg_sh.at[pl.ds(nbase, ROWS_PER_TILE)])
    plsc.subcore_barrier()

    @pl.loop(0, CHUNKS_PER_TILE)
    def _(j):
        pltpu.async_copy(ones_v, deg_sh.at[dst_v.at[j]], ssem, add=True)

    @pl.loop(0, CHUNKS_PER_TILE)
    def _(j):
        pltpu.make_async_copy(ones_v, deg_sh.at[dst_v.at[j]], ssem).wait()

    plsc.subcore_barrier()
    pltpu.sync_copy(deg_sh.at[pl.ds(nbase, ROWS_PER_TILE)], out_hbm.at[wid])


def _scale_body(deg_ref, x_ref, dis_ref, y_ref):
    d = deg_ref[0, 0, :, 0:1] + deg_ref[0, 1, :, 0:1] + 1.0
    di = lax.rsqrt(d)
    dis_ref[...] = di
    y_ref[...] = x_ref[...] * di


def _tc_scale(deg4, x_pad):
    return pl.pallas_call(
        _scale_body,
        grid=(NS,),
        in_specs=[
            pl.BlockSpec((1, NC, ROWS_PER_TILE, DEG_W), lambda i: (i, 0, 0, 0)),
            pl.BlockSpec((ROWS_PER_TILE, D_IN), lambda i: (i, 0)),
        ],
        out_specs=[
            pl.BlockSpec((ROWS_PER_TILE, 1), lambda i: (i, 0)),
            pl.BlockSpec((ROWS_PER_TILE, D_IN), lambda i: (i, 0)),
        ],
        out_shape=[
            jax.ShapeDtypeStruct((N_PAD, 1), jnp.float32),
            jax.ShapeDtypeStruct((N_PAD, D_IN), jnp.float32),
        ],
    )(deg4, x_pad)


GCH = 128
NCHUNKS = E_PAD // GCH
NBUF = 2
GIB = 16
G0 = 80
G1 = NCHUNKS // NS - G0


@functools.partial(
    pl.kernel,
    out_type=jax.ShapeDtypeStruct((NW, ROWS_PER_TILE, D_IN), jnp.float32),
    mesh=_mesh,
    scratch_types=[
        pltpu.VMEM((GIB, GCH), jnp.int32),
        pltpu.VMEM((GIB, GCH), jnp.int32),
        [pltpu.VMEM((GCH, D_IN), jnp.float32) for _ in range(NBUF)],
        pltpu.VMEM_SHARED((N_PAD, D_IN), jnp.float32),
        [pltpu.SemaphoreType.DMA for _ in range(NBUF)],
        [pltpu.SemaphoreType.DMA for _ in range(NBUF)],
    ],
)
def _sc_aggregate(src_hbm, dst_hbm, y_hbm, zeros_hbm, out_hbm,
                  src_b, dst_b, bufs, s_sh, sems, ssems):
    cid = lax.axis_index("c")
    sid = lax.axis_index("s")
    wid = sid * NC + cid
    ebase = jnp.where(cid == 0, sid * G0, NS * G0 + sid * G1)
    nrefill = jnp.where(cid == 0, G0 // GIB, G1 // GIB)
    nbase = sid * ROWS_PER_TILE
    with jax.named_scope("agg_init"):
        pltpu.sync_copy(zeros_hbm.at[pl.ds(nbase, ROWS_PER_TILE)],
                        s_sh.at[pl.ds(nbase, ROWS_PER_TILE)])
        plsc.subcore_barrier()

    with jax.named_scope("agg_edges"):
        @pl.loop(0, nrefill)
        def _(b):
            cbase = ebase + b * GIB
            pltpu.sync_copy(src_hbm.at[pl.ds(cbase, GIB)], src_b)
            pltpu.sync_copy(dst_hbm.at[pl.ds(cbase, GIB)], dst_b)
            for k in range(NBUF - 1):
                pltpu.async_copy(y_hbm.at[src_b.at[k]], bufs[k], sems[k])

            @pl.loop(0, GIB // NBUF)
            def _(t):
                for k in range(NBUF):
                    j = NBUF * t + k
                    jn = j + NBUF - 1
                    kn = (k + NBUF - 1) % NBUF

                    @pl.when(jn < GIB)
                    def _():
                        @pl.when(jn >= NBUF)
                        def _():
                            pltpu.make_async_copy(
                                bufs[kn], s_sh.at[dst_b.at[jn - NBUF]],
                                ssems[kn]).wait()

                        pltpu.async_copy(y_hbm.at[src_b.at[jn]], bufs[kn],
                                         sems[kn])

                    pltpu.make_async_copy(y_hbm.at[src_b.at[j]], bufs[k],
                                          sems[k]).wait()
                    pltpu.async_copy(bufs[k], s_sh.at[dst_b.at[j]], ssems[k],
                                     add=True)

            for k in range(NBUF):
                pltpu.make_async_copy(bufs[k], s_sh.at[dst_b.at[k]],
                                      ssems[k]).wait()

    with jax.named_scope("agg_drain"):
        plsc.subcore_barrier()
        pltpu.sync_copy(s_sh.at[pl.ds(nbase, ROWS_PER_TILE)], out_hbm.at[wid])


def _mlp_body(s_ref, y_ref, dis_ref, wc_ref, bc_ref, w1_ref, b1_ref,
              w2_ref, b2_ref, out_ref):
    z = (s_ref[0, 0] + s_ref[0, 1] + y_ref[...]) * dis_ref[...]
    h = jnp.dot(z, wc_ref[...], preferred_element_type=jnp.float32,
                precision=lax.Precision.DEFAULT) + bc_ref[...]
    h = jnp.maximum(h, 0.0)
    h = jnp.dot(h, w1_ref[...], preferred_element_type=jnp.float32,
                precision=lax.Precision.DEFAULT) + b1_ref[...]
    h = jnp.maximum(h, 0.0)
    out_ref[...] = jnp.dot(h, w2_ref[...], preferred_element_type=jnp.float32,
                           precision=lax.Precision.DEFAULT) + b2_ref[...]


def _tc_mlp(s4, y, dis, W_conv, b_conv, W_fc1, b_fc1, W_fc2, b_fc2):
    full = lambda shape: pl.BlockSpec(shape, lambda i: tuple(0 for _ in shape))
    return pl.pallas_call(
        _mlp_body,
        grid=(NS,),
        in_specs=[
            pl.BlockSpec((1, NC, ROWS_PER_TILE, D_IN), lambda i: (i, 0, 0, 0)),
            pl.BlockSpec((ROWS_PER_TILE, D_IN), lambda i: (i, 0)),
            pl.BlockSpec((ROWS_PER_TILE, 1), lambda i: (i, 0)),
            full((D_IN, D1)),
            full((1, D1)),
            full((D1, D2)),
            full((1, D2)),
            full((D2, D_IN)),
            full((1, D_IN)),
        ],
        out_specs=pl.BlockSpec((ROWS_PER_TILE, D_IN), lambda i: (i, 0)),
        out_shape=jax.ShapeDtypeStruct((N, D_IN), jnp.float32),
    )(s4, y, dis, W_conv, b_conv, W_fc1, b_fc1, W_fc2, b_fc2)


def kernel(x, edge_index, W_conv, b_conv, W_fc1, b_fc1, W_fc2, b_fc2):
    ei = edge_index.astype(jnp.int32)
    pad_e = E_PAD - E
    src = jnp.concatenate(
        [ei[0], (jnp.arange(pad_e, dtype=jnp.int32) * 32) % N])
    dst = jnp.concatenate([ei[1], jnp.full((pad_e,), N_PAD - 1, jnp.int32)])
    dst2d = dst.reshape(-1, CHUNK)
    src2g = src.reshape(-1, GCH)
    dst2g = dst.reshape(-1, GCH)
    x_pad = jnp.concatenate(
        [x, jnp.zeros((N_PAD - N, D_IN), jnp.float32)], axis=0)

    zeros_nd = jnp.zeros((N_PAD, D_IN), jnp.float32)
    ones_w = jnp.ones((CHUNK, DEG_W), jnp.float32)
    deg_parts = _sc_degree(dst2d, zeros_nd, ones_w)
    deg4 = deg_parts.reshape(NS, NC, ROWS_PER_TILE, DEG_W)

    dis, y = _tc_scale(deg4, x_pad)

    s_parts = _sc_aggregate(src2g, dst2g, y, zeros_nd)
    s4 = s_parts.reshape(NS, NC, ROWS_PER_TILE, D_IN)

    return _tc_mlp(s4, y, dis, W_conv, b_conv.reshape(1, D1),
                   W_fc1, b_fc1.reshape(1, D2), W_fc2, b_fc2.reshape(1, D_IN))
